# baseline (speedup 1.0000x reference)
import numpy as np
import jax
import jax.numpy as jnp
from jax.experimental import pallas as pl

_EMB_DIM = 128
_EDGE_CONT = {0: [3, 6, 7, 9, 10, 11, 12, 13], 1: [2, 3, 4, 5, 6, 7, 9, 10, 11, 12, 13], 2: [2, 3, 4, 5, 6, 7, 9, 10, 11, 12, 13], 3: [1, 4, 5, 6, 7, 9, 10, 11, 12, 13], 4: [2, 3, 4, 5, 6, 7, 9, 10, 11, 12, 13], 5: [1, 2, 3, 4, 5, 6, 7, 9, 10, 11, 12, 13], 6: [2, 3, 4, 5, 6, 7, 9, 10, 11, 12, 13], 7: [1, 2, 3, 4, 5, 6, 7, 9, 10, 11, 12, 13], 8: [0, 1, 4, 6, 7, 9, 10, 11, 12, 13]}
_EDGE_DISC_FEATS = {0: [0, 1, 2, 4, 5, 8], 1: [0, 1, 8], 2: [0, 1, 8], 3: [0, 2, 3, 8], 4: [0, 1, 8], 5: [0, 8], 6: [0, 1, 8], 7: [0, 8], 8: [2, 3, 5, 8]}
_COL_RANGES = {0: 4, 1: 6, 2: 6, 3: 8, 4: 15, 5: 2, 8: 9}

_K = 256
_HALF = 128
_BLOCK = 6400

_CONT_LANES = []
for _t in range(9):
    for _c in _EDGE_CONT[_t]:
        _CONT_LANES.append((_c, _t))
_DISC_LANES = []
for _c in [0, 1, 2, 3, 4, 5]:
    for _t in range(9):
        if _c in _EDGE_DISC_FEATS[_t]:
            for _v in range(_COL_RANGES[_c]):
                _DISC_LANES.append((_c, _t, _v))
for _v in range(9):
    _DISC_LANES.append((8, _v, _v))
_NC = len(_CONT_LANES)
_ND = len(_DISC_LANES)
assert _NC + _ND <= _K and _NC <= _HALF

_SS = np.zeros((16, _K + _HALF), np.float32)
_SS[14, :_K] = -1.0
for _i, (_c, _t) in enumerate(_CONT_LANES):
    _SS[8, _i] = 16.0
    _SS[14, _i] = -16.0 * _t
    _SS[_c, _K + _i] = 1.0
for _j, (_c, _t, _v) in enumerate(_DISC_LANES):
    _L = _NC + _j
    _SS[_c, _L] = 1.0 + (16.0 if _c == 8 else 0.0)
    if _c != 8:
        _SS[8, _L] = 16.0
    _SS[14, _L] = -(_v + 16.0 * _t)
    if _L < _HALF:
        _SS[14, _K + _L] = 1.0


def _col_spans(t):
    feats = _EDGE_DISC_FEATS[t]
    nd = len(feats)
    per, rem = _EMB_DIM // nd, _EMB_DIM % nd
    spans, col = {}, 0
    for i, f in enumerate(feats):
        dim = per + (1 if i < rem else 0)
        spans[f] = (col, dim)
        col += dim
    return spans


def _build_g(params):
    spans = {t: _col_spans(t) for t in range(9)}
    pieces = [params["W"][str(t)].T for t in range(9)]
    i = 0
    while i < _ND:
        c, t, v = _DISC_LANES[i]
        lo, dim = spans[t][c]
        if c != 8:
            r = _COL_RANGES[c]
            tbl = params["tables"][str(t)][str(c)][:r, :]
            pieces.append(jnp.pad(tbl, ((0, 0), (lo, _EMB_DIM - lo - dim))))
            i += r
        else:
            row = jnp.pad(params["tables"][str(t)]["8"][v:v + 1, :],
                          ((0, 0), (lo, _EMB_DIM - lo - dim)))
            pieces.append(row + params["b"][str(t)][None, :])
            i += 1
    pieces.append(jnp.zeros((_K - _NC - _ND, _EMB_DIM), jnp.float32))
    return jnp.concatenate(pieces, axis=0).astype(jnp.bfloat16)


def _body(a_ref, ss_ref, g_ref, o_ref):
    a = a_ref[:, :]
    b = a.shape[0]
    az = jnp.concatenate(
        [a, jnp.ones((b, 1), jnp.float32), jnp.zeros((b, 1), jnp.float32)],
        axis=1).astype(jnp.bfloat16)
    mm = jnp.dot(az, ss_ref[:, :], preferred_element_type=jnp.float32)
    hit = mm[:, :_K] == 0.0
    lo = jnp.where(hit[:, :_HALF], mm[:, _K:], 0.0).astype(jnp.bfloat16)
    hi = hit[:, _HALF:].astype(jnp.bfloat16)
    phi = jnp.concatenate([lo, hi], axis=1)
    o_ref[:, :] = jnp.dot(phi, g_ref[:, :],
                          preferred_element_type=jnp.float32)


def kernel(edge_attr, params):
    n = edge_attr.shape[0]
    g = _build_g(params)
    grid = n // _BLOCK
    return pl.pallas_call(
        _body,
        grid=(grid,),
        in_specs=[
            pl.BlockSpec((_BLOCK, 14), lambda i: (i, 0)),
            pl.BlockSpec((16, _K + _HALF), lambda i: (0, 0)),
            pl.BlockSpec((_K, _EMB_DIM), lambda i: (0, 0)),
        ],
        out_specs=pl.BlockSpec((_BLOCK, _EMB_DIM), lambda i: (i, 0)),
        out_shape=jax.ShapeDtypeStruct((n, _EMB_DIM), jnp.float32),
    )(edge_attr, jnp.asarray(_SS, jnp.bfloat16), g)
